# baseline (speedup 1.0000x reference)
import jax
import jax.numpy as jnp
from jax import lax
from jax.experimental import pallas as pl
from jax.experimental.pallas import tpu as pltpu
from jax.experimental.pallas import tpu_sc as plsc

N = 10000
E = 160000
D = 256
DE = 16
G = 64

NPAD = 10240
EPAD = 163840
NC = 2
NS = 16
NW = NC * NS
NPS = NPAD // NS
ECHUNK = EPAD // NW
EB = 2048
XB = 1024

_HIGH = lax.Precision.HIGHEST


def _xl1_body(x_ref, w_ref, o_ref):
    o_ref[...] = lax.dot_general(w_ref[...], x_ref[...],
                                 (((0,), (1,)), ((), ())),
                                 precision=_HIGH)


def _edge_w_body(ea_ref, m1w1, m1b1, m1w2, m1b2, m2w1, m2b1, m2w2, m2b2,
                 w1_ref, w2_ref):
    i = pl.program_id(0)
    e_glob = i * EB + lax.broadcasted_iota(jnp.int32, (1, EB), 1)
    valid = e_glob < E
    ea = ea_ref[...]

    def mlp(w1, b1, w2, b2):
        h = lax.dot_general(w1[...], ea, (((0,), (1,)), ((), ())),
                            precision=_HIGH)
        h = jnp.maximum(h + b1[...], 0.0)
        o = lax.dot_general(w2[...], h, (((0,), (0,)), ((), ())),
                            precision=_HIGH)
        return jax.nn.sigmoid(o + b2[...])

    w1_ref[...] = jnp.where(valid, mlp(m1w1, m1b1, m1w2, m1b2), 0.0)
    w2_ref[...] = jnp.where(valid, mlp(m2w1, m2b1, m2w2, m2b2), 0.0)


def _mid_body(msg_ref, deg_ref, xlt_ref, batch_ref, b1_ref, gnw_ref, gnb_ref,
              gnms_ref, w2_ref, out_ref):
    h = (msg_ref[0] + msg_ref[1]
         + xlt_ref[...] / deg_ref[...]
         + b1_ref[...])
    ohT = (lax.broadcasted_iota(jnp.int32, (G, NPAD), 0)
           == batch_ref[...]).astype(jnp.float32)
    cnt = jnp.maximum(jnp.sum(ohT, axis=1), 1.0)[None, :]
    seg = lax.dot_general(h, ohT, (((1,), (1,)), ((), ())),
                          precision=_HIGH)
    mean = seg / cnt
    mean_b = lax.dot_general(mean, ohT, (((1,), (0,)), ((), ())),
                             precision=_HIGH)
    out = h - mean_b * gnms_ref[...]
    var = lax.dot_general(out * out, ohT, (((1,), (1,)), ((), ())),
                          precision=_HIGH) / cnt
    std = jnp.sqrt(var + 1e-5)
    std_b = lax.dot_general(std, ohT, (((1,), (0,)), ((), ())),
                            precision=_HIGH)
    std_b = jnp.where(std_b > 0.0, std_b, 1.0)
    hn = gnw_ref[...] * out / std_b + gnb_ref[...]
    hr = jnp.maximum(hn, 0.0)
    out_ref[...] = lax.dot_general(w2_ref[...], hr, (((0,), (0,)), ((), ())),
                                   precision=_HIGH)


def _final_body(msg_ref, deg_ref, xl2_ref, b2_ref, out_ref):
    h = (msg_ref[pl.ds(0, 1), :] + msg_ref[pl.ds(1, 1), :]
         + xl2_ref[...] / deg_ref[...] + b2_ref[...])
    out_ref[...] = jax.nn.sigmoid(h)


def _rsqrt16(x):
    i = plsc.bitcast(x, jnp.int32)
    i = jnp.int32(0x5F3759DF) - lax.shift_right_logical(i, 1)
    y = plsc.bitcast(i, jnp.float32)
    hx = 0.5 * x
    for _ in range(4):
        y = y * (1.5 - (hx * y) * y)
    return y


def _sc_msg1_body(row_h, col_h, w1_h, w2_h, xlt_h,
                  msg_o, deg1_o, deg2_o, wn2_o,
                  rowb, colb, w1b, w2b, wn1b, wn2b, valb,
                  dis1l, dis2l, xll, nodeb,
                  deg1s, deg2s, dis1s, dis2s, m0s, m1s, m2s, m3s, m4s):
    c = lax.axis_index("c")
    s = lax.axis_index("s")
    wid = c * NS + s
    nbase = s * NPS
    msgs = (m0s, m1s, m2s, m3s, m4s)

    def zero_loop(i, _):
        nodeb[pl.ds(i * 16, 16)] = jnp.zeros((16,), jnp.float32)
        return 0
    lax.fori_loop(0, NPS // 16, zero_loop, 0)
    for m in msgs:
        pltpu.sync_copy(nodeb, m.at[pl.ds(nbase, NPS)])

    def one_loop(i, _):
        nodeb[pl.ds(i * 16, 16)] = jnp.ones((16,), jnp.float32)
        return 0
    lax.fori_loop(0, NPS // 16, one_loop, 0)
    pltpu.sync_copy(nodeb, deg1s.at[pl.ds(nbase, NPS)])
    pltpu.sync_copy(nodeb, deg2s.at[pl.ds(nbase, NPS)])
    plsc.subcore_barrier()

    for h in range(2):
        dbase = s * (2 * ECHUNK) + h * ECHUNK
        pltpu.sync_copy(col_h.at[pl.ds(dbase, ECHUNK)], colb)
        pltpu.sync_copy(w1_h.at[pl.ds(dbase, ECHUNK)], w1b)
        pltpu.sync_copy(w1b, deg1s.at[colb], add=True)
        pltpu.sync_copy(w2_h.at[pl.ds(dbase, ECHUNK)], w2b)
        pltpu.sync_copy(w2b, deg2s.at[colb], add=True)
    plsc.subcore_barrier()

    @pl.when(c == 0)
    def _():
        pltpu.sync_copy(deg1s.at[pl.ds(nbase, NPS)],
                        deg1_o.at[pl.ds(nbase, NPS)])
        pltpu.sync_copy(deg2s.at[pl.ds(nbase, NPS)],
                        deg2_o.at[pl.ds(nbase, NPS)])

    for deg_s, dis_s in ((deg1s, dis1s), (deg2s, dis2s)):
        pltpu.sync_copy(deg_s.at[pl.ds(nbase, NPS)], nodeb)

        def rs_loop(i, _):
            sl = pl.ds(i * 16, 16)
            nodeb[sl] = _rsqrt16(nodeb[sl])
            return 0
        lax.fori_loop(0, NPS // 16, rs_loop, 0)
        pltpu.sync_copy(nodeb, dis_s.at[pl.ds(nbase, NPS)])
    plsc.subcore_barrier()

    pltpu.sync_copy(dis1s, dis1l)
    pltpu.sync_copy(dis2s, dis2l)
    pltpu.sync_copy(xlt_h, xll)

    ebase = wid * ECHUNK
    pltpu.sync_copy(row_h.at[pl.ds(ebase, ECHUNK)], rowb)
    pltpu.sync_copy(col_h.at[pl.ds(ebase, ECHUNK)], colb)
    pltpu.sync_copy(w1_h.at[pl.ds(ebase, ECHUNK)], w1b)
    pltpu.sync_copy(w2_h.at[pl.ds(ebase, ECHUNK)], w2b)

    def wn_loop(i, _):
        sl = pl.ds(i * 16, 16)
        r = rowb[sl]
        cc = colb[sl]
        wn1b[sl] = (w1b[sl] * plsc.load_gather(dis1l, [r])) \
            * plsc.load_gather(dis1l, [cc])
        wn2b[sl] = (w2b[sl] * plsc.load_gather(dis2l, [r])) \
            * plsc.load_gather(dis2l, [cc])
        return 0
    lax.fori_loop(0, ECHUNK // 16, wn_loop, 0)
    pltpu.sync_copy(wn2b, wn2_o.at[pl.ds(ebase, ECHUNK)])

    for f in range(5):
        foff = jnp.int32(f * NPAD)

        def msg_loop(i, _):
            sl = pl.ds(i * 16, 16)
            valb[sl] = wn1b[sl] * plsc.load_gather(xll, [rowb[sl] + foff])
            return 0
        lax.fori_loop(0, ECHUNK // 16, msg_loop, 0)
        pltpu.sync_copy(valb, msgs[f].at[colb], add=True)
    plsc.subcore_barrier()

    for f in range(5):
        moff = (c * 5 + f) * NPAD + nbase
        pltpu.sync_copy(msgs[f].at[pl.ds(nbase, NPS)],
                        msg_o.at[pl.ds(moff, NPS)])


def _sc_msg2_body(row_h, col_h, wn_h, xl2_h,
                  msg_o,
                  rowb, colb, wnb, valb, xl2l, nodeb,
                  m0s):
    c = lax.axis_index("c")
    s = lax.axis_index("s")
    wid = c * NS + s
    nbase = s * NPS

    def zero_loop(i, _):
        nodeb[pl.ds(i * 16, 16)] = jnp.zeros((16,), jnp.float32)
        return 0
    lax.fori_loop(0, NPS // 16, zero_loop, 0)
    pltpu.sync_copy(nodeb, m0s.at[pl.ds(nbase, NPS)])
    plsc.subcore_barrier()

    pltpu.sync_copy(xl2_h, xl2l)
    ebase = wid * ECHUNK
    pltpu.sync_copy(row_h.at[pl.ds(ebase, ECHUNK)], rowb)
    pltpu.sync_copy(col_h.at[pl.ds(ebase, ECHUNK)], colb)
    pltpu.sync_copy(wn_h.at[pl.ds(ebase, ECHUNK)], wnb)

    def msg_loop(i, _):
        sl = pl.ds(i * 16, 16)
        valb[sl] = wnb[sl] * plsc.load_gather(xl2l, [rowb[sl]])
        return 0
    lax.fori_loop(0, ECHUNK // 16, msg_loop, 0)
    pltpu.sync_copy(valb, m0s.at[colb], add=True)
    plsc.subcore_barrier()

    pltpu.sync_copy(m0s.at[pl.ds(nbase, NPS)],
                    msg_o.at[pl.ds(c * NPAD + nbase, NPS)])


def _sc_mesh():
    return plsc.VectorSubcoreMesh(core_axis_name="c", subcore_axis_name="s",
                                  num_cores=NC, num_subcores=NS)


_full_spec = lambda shp: pl.BlockSpec(shp, lambda: tuple(0 for _ in shp))


@jax.jit
def kernel(x, edge_index, edge_attr, batch_idx, W1, b1, mlp1_w1, mlp1_b1,
           mlp1_w2, mlp1_b2, gn_w, gn_b, gn_ms, W2, b2, mlp2_w1, mlp2_b1,
           mlp2_w2, mlp2_b2):
    f32 = jnp.float32

    x_p = jnp.pad(x, ((0, NPAD - N), (0, 0)))
    row_p = jnp.pad(edge_index[0], (0, EPAD - E), constant_values=NPAD - 1)
    col_p = jnp.pad(edge_index[1], (0, EPAD - E), constant_values=NPAD - 1)
    ea_p = jnp.pad(edge_attr, ((0, EPAD - E), (0, 0)))
    batch_p = jnp.pad(batch_idx, (0, NPAD - N), constant_values=-1)[None, :]

    xlt = pl.pallas_call(
        _xl1_body,
        grid=(NPAD // XB,),
        in_specs=[pl.BlockSpec((XB, D), lambda i: (i, 0)),
                  pl.BlockSpec((D, 5), lambda i: (0, 0))],
        out_specs=pl.BlockSpec((5, XB), lambda i: (0, i)),
        out_shape=jax.ShapeDtypeStruct((5, NPAD), f32),
    )(x_p, W1)

    wspec = pl.BlockSpec((1, EB), lambda i: (0, i))
    full = lambda shp: pl.BlockSpec(shp, lambda i: tuple(0 for _ in shp))
    w1e, w2e = pl.pallas_call(
        _edge_w_body,
        grid=(EPAD // EB,),
        in_specs=[pl.BlockSpec((EB, DE), lambda i: (i, 0)),
                  full((DE, 5)), full((5, 1)), full((5, 1)), full((1, 1)),
                  full((DE, 5)), full((5, 1)), full((5, 1)), full((1, 1))],
        out_specs=[wspec, wspec],
        out_shape=[jax.ShapeDtypeStruct((1, EPAD), f32),
                   jax.ShapeDtypeStruct((1, EPAD), f32)],
    )(ea_p, mlp1_w1, mlp1_b1[:, None], mlp1_w2, mlp1_b2[:, None],
      mlp2_w1, mlp2_b1[:, None], mlp2_w2, mlp2_b2[:, None])
    w1e = w1e.reshape(EPAD)
    w2e = w2e.reshape(EPAD)

    sc1 = pl.kernel(
        _sc_msg1_body,
        out_type=[jax.ShapeDtypeStruct((NC * 5 * NPAD,), f32),
                  jax.ShapeDtypeStruct((NPAD,), f32),
                  jax.ShapeDtypeStruct((NPAD,), f32),
                  jax.ShapeDtypeStruct((EPAD,), f32)],
        mesh=_sc_mesh(),
        compiler_params=pltpu.CompilerParams(needs_layout_passes=False),
        scratch_types=[
            pltpu.VMEM((ECHUNK,), jnp.int32),
            pltpu.VMEM((ECHUNK,), jnp.int32),
            pltpu.VMEM((ECHUNK,), f32),
            pltpu.VMEM((ECHUNK,), f32),
            pltpu.VMEM((ECHUNK,), f32),
            pltpu.VMEM((ECHUNK,), f32),
            pltpu.VMEM((ECHUNK,), f32),
            pltpu.VMEM((NPAD,), f32),
            pltpu.VMEM((NPAD,), f32),
            pltpu.VMEM((5 * NPAD,), f32),
            pltpu.VMEM((NPS,), f32),
            pltpu.VMEM_SHARED((NPAD,), f32),
            pltpu.VMEM_SHARED((NPAD,), f32),
            pltpu.VMEM_SHARED((NPAD,), f32),
            pltpu.VMEM_SHARED((NPAD,), f32),
            pltpu.VMEM_SHARED((NPAD,), f32),
            pltpu.VMEM_SHARED((NPAD,), f32),
            pltpu.VMEM_SHARED((NPAD,), f32),
            pltpu.VMEM_SHARED((NPAD,), f32),
            pltpu.VMEM_SHARED((NPAD,), f32),
        ],
    )
    msg1, deg1, deg2, wn2 = sc1(row_p, col_p, w1e, w2e, xlt.reshape(5 * NPAD))
    msg1 = msg1.reshape(NC, 5, NPAD)
    deg1 = deg1[None, :]
    deg2 = deg2[None, :]

    xl2 = pl.pallas_call(
        _mid_body,
        in_specs=[
            _full_spec((NC, 5, NPAD)),
            _full_spec((1, NPAD)),
            _full_spec((5, NPAD)),
            _full_spec((1, NPAD)),
            _full_spec((5, 1)),
            _full_spec((5, 1)),
            _full_spec((5, 1)),
            _full_spec((5, 1)),
            _full_spec((5, 1)),
        ],
        out_specs=_full_spec((1, NPAD)),
        out_shape=jax.ShapeDtypeStruct((1, NPAD), f32),
    )(msg1, deg1, xlt, batch_p, b1[:, None], gn_w[:, None],
      gn_b[:, None], gn_ms[:, None], W2)

    sc2 = pl.kernel(
        _sc_msg2_body,
        out_type=[jax.ShapeDtypeStruct((NC * NPAD,), f32)],
        mesh=_sc_mesh(),
        compiler_params=pltpu.CompilerParams(needs_layout_passes=False),
        scratch_types=[
            pltpu.VMEM((ECHUNK,), jnp.int32),
            pltpu.VMEM((ECHUNK,), jnp.int32),
            pltpu.VMEM((ECHUNK,), f32),
            pltpu.VMEM((ECHUNK,), f32),
            pltpu.VMEM((NPAD,), f32),
            pltpu.VMEM((NPS,), f32),
            pltpu.VMEM_SHARED((NPAD,), f32),
        ],
    )
    (msg2,) = sc2(row_p, col_p, wn2, xl2.reshape(NPAD))
    msg2 = msg2.reshape(NC, NPAD)

    out = pl.pallas_call(
        _final_body,
        in_specs=[_full_spec((NC, NPAD)), _full_spec((1, NPAD)),
                  _full_spec((1, NPAD)), _full_spec((1, 1))],
        out_specs=_full_spec((1, NPAD)),
        out_shape=jax.ShapeDtypeStruct((1, NPAD), f32),
    )(msg2, deg2, xl2, b2[:, None])

    return out[0, :N, None]


# speedup vs baseline: 1.6730x; 1.6730x over previous
import jax
import jax.numpy as jnp
from jax import lax
from jax.experimental import pallas as pl
from jax.experimental.pallas import tpu as pltpu
from jax.experimental.pallas import tpu_sc as plsc

N = 10000
E = 160000
D = 256
DE = 16
G = 64

NPAD = 10240
EPAD = 163840
NC = 2
NS = 16
NW = NC * NS
NPS = NPAD // NS
ECHUNK = EPAD // NW
EB = 16384
XB = 1024

_HIGH = lax.Precision.HIGHEST


def _xl1_body(x_ref, w_ref, o_ref):
    i = pl.program_id(0)
    n_glob = i * XB + lax.broadcasted_iota(jnp.int32, (1, XB), 1)
    o = lax.dot_general(w_ref[...], x_ref[...],
                        (((0,), (1,)), ((), ())),
                        precision=_HIGH)
    o_ref[...] = jnp.where(n_glob < N, o, 0.0)


def _edge_w_body(ea_ref, m1w1, m1b1, m1w2, m1b2, m2w1, m2b1, m2w2, m2b2,
                 w1_ref, w2_ref):
    i = pl.program_id(0)
    e_glob = i * EB + lax.broadcasted_iota(jnp.int32, (1, EB), 1)
    valid = e_glob < E
    ea = ea_ref[...]

    def mlp(w1, b1, w2, b2):
        h = lax.dot_general(w1[...], ea, (((0,), (1,)), ((), ())))
        h = jnp.maximum(h + b1[...], 0.0)
        o = lax.dot_general(w2[...], h, (((0,), (0,)), ((), ())))
        return jax.nn.sigmoid(o + b2[...])

    w1_ref[...] = jnp.where(valid, mlp(m1w1, m1b1, m1w2, m1b2), 0.0)
    w2_ref[...] = jnp.where(valid, mlp(m2w1, m2b1, m2w2, m2b2), 0.0)


def _mid_body(msg_ref, deg_ref, xlt_ref, batch_ref, b1_ref, gnw_ref, gnb_ref,
              gnms_ref, w2_ref, out_ref):
    h = (msg_ref[0] + msg_ref[1]
         + xlt_ref[...] / deg_ref[...]
         + b1_ref[...])
    ohT = (lax.broadcasted_iota(jnp.int32, (G, NPAD), 0)
           == batch_ref[...]).astype(jnp.float32)
    cnt = jnp.maximum(jnp.sum(ohT, axis=1), 1.0)[None, :]
    seg = lax.dot_general(h, ohT, (((1,), (1,)), ((), ())),
                          precision=_HIGH)
    mean = seg / cnt
    mean_b = lax.dot_general(mean, ohT, (((1,), (0,)), ((), ())),
                             precision=_HIGH)
    out = h - mean_b * gnms_ref[...]
    var = lax.dot_general(out * out, ohT, (((1,), (1,)), ((), ())),
                          precision=_HIGH) / cnt
    std = jnp.sqrt(var + 1e-5)
    std_b = lax.dot_general(std, ohT, (((1,), (0,)), ((), ())),
                            precision=_HIGH)
    std_b = jnp.where(std_b > 0.0, std_b, 1.0)
    hn = gnw_ref[...] * out / std_b + gnb_ref[...]
    hr = jnp.maximum(hn, 0.0)
    out_ref[...] = lax.dot_general(w2_ref[...], hr, (((0,), (0,)), ((), ())),
                                   precision=_HIGH)


def _final_body(msg_ref, deg_ref, xl2_ref, b2_ref, out_ref):
    h = (msg_ref[pl.ds(0, 1), :] + msg_ref[pl.ds(1, 1), :]
         + xl2_ref[...] / deg_ref[...] + b2_ref[...])
    out_ref[...] = jax.nn.sigmoid(h)


def _rsqrt16(x):
    i = plsc.bitcast(x, jnp.int32)
    i = jnp.int32(0x5F3759DF) - lax.shift_right_logical(i, 1)
    y = plsc.bitcast(i, jnp.float32)
    hx = 0.5 * x
    for _ in range(4):
        y = y * (1.5 - (hx * y) * y)
    return y


def _sc_msg1_body(row_h, col_h, w1_h, w2_h, xlt_h,
                  msg_o, deg1_o, deg2_o, wn2_o,
                  rowb, colb, w1b, w2b, wn1b, wn2b, valb,
                  dis1l, dis2l, xll, nodeb,
                  deg1s, deg2s, dis1s, dis2s, m0s, m1s, m2s, m3s, m4s):
    c = lax.axis_index("c")
    s = lax.axis_index("s")
    wid = c * NS + s
    nbase = s * NPS
    msgs = (m0s, m1s, m2s, m3s, m4s)

    def zero_loop(i, _):
        nodeb[pl.ds(i * 16, 16)] = jnp.zeros((16,), jnp.float32)
        return 0
    lax.fori_loop(0, NPS // 16, zero_loop, 0)
    for m in msgs:
        pltpu.sync_copy(nodeb, m.at[pl.ds(nbase, NPS)])

    def one_loop(i, _):
        nodeb[pl.ds(i * 16, 16)] = jnp.ones((16,), jnp.float32)
        return 0
    lax.fori_loop(0, NPS // 16, one_loop, 0)
    pltpu.sync_copy(nodeb, deg1s.at[pl.ds(nbase, NPS)])
    pltpu.sync_copy(nodeb, deg2s.at[pl.ds(nbase, NPS)])
    plsc.subcore_barrier()

    for h in range(2):
        dbase = s * (2 * ECHUNK) + h * ECHUNK
        pltpu.sync_copy(col_h.at[pl.ds(dbase, ECHUNK)], colb)
        pltpu.sync_copy(w1_h.at[pl.ds(dbase, ECHUNK)], w1b)
        pltpu.sync_copy(w1b, deg1s.at[colb], add=True)
        pltpu.sync_copy(w2_h.at[pl.ds(dbase, ECHUNK)], w2b)
        pltpu.sync_copy(w2b, deg2s.at[colb], add=True)
    plsc.subcore_barrier()

    @pl.when(c == 0)
    def _():
        pltpu.sync_copy(deg1s.at[pl.ds(nbase, NPS)],
                        deg1_o.at[pl.ds(nbase, NPS)])
        pltpu.sync_copy(deg2s.at[pl.ds(nbase, NPS)],
                        deg2_o.at[pl.ds(nbase, NPS)])

    for deg_s, dis_s in ((deg1s, dis1s), (deg2s, dis2s)):
        pltpu.sync_copy(deg_s.at[pl.ds(nbase, NPS)], nodeb)

        def rs_loop(i, _):
            sl = pl.ds(i * 16, 16)
            nodeb[sl] = _rsqrt16(nodeb[sl])
            return 0
        lax.fori_loop(0, NPS // 16, rs_loop, 0)
        pltpu.sync_copy(nodeb, dis_s.at[pl.ds(nbase, NPS)])
    plsc.subcore_barrier()

    pltpu.sync_copy(dis1s, dis1l)
    pltpu.sync_copy(dis2s, dis2l)
    pltpu.sync_copy(xlt_h, xll)

    ebase = wid * ECHUNK
    pltpu.sync_copy(row_h.at[pl.ds(ebase, ECHUNK)], rowb)
    pltpu.sync_copy(col_h.at[pl.ds(ebase, ECHUNK)], colb)
    pltpu.sync_copy(w1_h.at[pl.ds(ebase, ECHUNK)], w1b)
    pltpu.sync_copy(w2_h.at[pl.ds(ebase, ECHUNK)], w2b)

    def wn_loop(i, _):
        sl = pl.ds(i * 16, 16)
        r = rowb[sl]
        cc = colb[sl]
        wn1b[sl] = (w1b[sl] * plsc.load_gather(dis1l, [r])) \
            * plsc.load_gather(dis1l, [cc])
        wn2b[sl] = (w2b[sl] * plsc.load_gather(dis2l, [r])) \
            * plsc.load_gather(dis2l, [cc])
        return 0
    lax.fori_loop(0, ECHUNK // 16, wn_loop, 0)
    pltpu.sync_copy(wn2b, wn2_o.at[pl.ds(ebase, ECHUNK)])

    for f in range(5):
        foff = jnp.int32(f * NPAD)

        def msg_loop(i, _):
            sl = pl.ds(i * 16, 16)
            valb[sl] = wn1b[sl] * plsc.load_gather(xll, [rowb[sl] + foff])
            return 0
        lax.fori_loop(0, ECHUNK // 16, msg_loop, 0)
        pltpu.sync_copy(valb, msgs[f].at[colb], add=True)
    plsc.subcore_barrier()

    for f in range(5):
        moff = (c * 5 + f) * NPAD + nbase
        pltpu.sync_copy(msgs[f].at[pl.ds(nbase, NPS)],
                        msg_o.at[pl.ds(moff, NPS)])


def _sc_msg2_body(row_h, col_h, wn_h, xl2_h,
                  msg_o,
                  rowb, colb, wnb, valb, xl2l, nodeb,
                  m0s):
    c = lax.axis_index("c")
    s = lax.axis_index("s")
    wid = c * NS + s
    nbase = s * NPS

    def zero_loop(i, _):
        nodeb[pl.ds(i * 16, 16)] = jnp.zeros((16,), jnp.float32)
        return 0
    lax.fori_loop(0, NPS // 16, zero_loop, 0)
    pltpu.sync_copy(nodeb, m0s.at[pl.ds(nbase, NPS)])
    plsc.subcore_barrier()

    pltpu.sync_copy(xl2_h, xl2l)
    ebase = wid * ECHUNK
    pltpu.sync_copy(row_h.at[pl.ds(ebase, ECHUNK)], rowb)
    pltpu.sync_copy(col_h.at[pl.ds(ebase, ECHUNK)], colb)
    pltpu.sync_copy(wn_h.at[pl.ds(ebase, ECHUNK)], wnb)

    def msg_loop(i, _):
        sl = pl.ds(i * 16, 16)
        valb[sl] = wnb[sl] * plsc.load_gather(xl2l, [rowb[sl]])
        return 0
    lax.fori_loop(0, ECHUNK // 16, msg_loop, 0)
    pltpu.sync_copy(valb, m0s.at[colb], add=True)
    plsc.subcore_barrier()

    pltpu.sync_copy(m0s.at[pl.ds(nbase, NPS)],
                    msg_o.at[pl.ds(c * NPAD + nbase, NPS)])


def _sc_mesh():
    return plsc.VectorSubcoreMesh(core_axis_name="c", subcore_axis_name="s",
                                  num_cores=NC, num_subcores=NS)


_full_spec = lambda shp: pl.BlockSpec(shp, lambda: tuple(0 for _ in shp))


@jax.jit
def kernel(x, edge_index, edge_attr, batch_idx, W1, b1, mlp1_w1, mlp1_b1,
           mlp1_w2, mlp1_b2, gn_w, gn_b, gn_ms, W2, b2, mlp2_w1, mlp2_b1,
           mlp2_w2, mlp2_b2):
    f32 = jnp.float32

    row_p = jnp.pad(edge_index[0], (0, EPAD - E), constant_values=NPAD - 1)
    col_p = jnp.pad(edge_index[1], (0, EPAD - E), constant_values=NPAD - 1)
    batch_p = jnp.pad(batch_idx, (0, NPAD - N), constant_values=-1)[None, :]

    xlt = pl.pallas_call(
        _xl1_body,
        grid=(NPAD // XB,),
        in_specs=[pl.BlockSpec((XB, D), lambda i: (i, 0)),
                  pl.BlockSpec((D, 5), lambda i: (0, 0))],
        out_specs=pl.BlockSpec((5, XB), lambda i: (0, i)),
        out_shape=jax.ShapeDtypeStruct((5, NPAD), f32),
    )(x, W1)

    wspec = pl.BlockSpec((1, EB), lambda i: (0, i))
    full = lambda shp: pl.BlockSpec(shp, lambda i: tuple(0 for _ in shp))
    w1e, w2e = pl.pallas_call(
        _edge_w_body,
        grid=(EPAD // EB,),
        in_specs=[pl.BlockSpec((EB, DE), lambda i: (i, 0)),
                  full((DE, 5)), full((5, 1)), full((5, 1)), full((1, 1)),
                  full((DE, 5)), full((5, 1)), full((5, 1)), full((1, 1))],
        out_specs=[wspec, wspec],
        out_shape=[jax.ShapeDtypeStruct((1, EPAD), f32),
                   jax.ShapeDtypeStruct((1, EPAD), f32)],
    )(edge_attr, mlp1_w1, mlp1_b1[:, None], mlp1_w2, mlp1_b2[:, None],
      mlp2_w1, mlp2_b1[:, None], mlp2_w2, mlp2_b2[:, None])
    w1e = w1e.reshape(EPAD)
    w2e = w2e.reshape(EPAD)

    sc1 = pl.kernel(
        _sc_msg1_body,
        out_type=[jax.ShapeDtypeStruct((NC * 5 * NPAD,), f32),
                  jax.ShapeDtypeStruct((NPAD,), f32),
                  jax.ShapeDtypeStruct((NPAD,), f32),
                  jax.ShapeDtypeStruct((EPAD,), f32)],
        mesh=_sc_mesh(),
        compiler_params=pltpu.CompilerParams(needs_layout_passes=False),
        scratch_types=[
            pltpu.VMEM((ECHUNK,), jnp.int32),
            pltpu.VMEM((ECHUNK,), jnp.int32),
            pltpu.VMEM((ECHUNK,), f32),
            pltpu.VMEM((ECHUNK,), f32),
            pltpu.VMEM((ECHUNK,), f32),
            pltpu.VMEM((ECHUNK,), f32),
            pltpu.VMEM((ECHUNK,), f32),
            pltpu.VMEM((NPAD,), f32),
            pltpu.VMEM((NPAD,), f32),
            pltpu.VMEM((5 * NPAD,), f32),
            pltpu.VMEM((NPS,), f32),
            pltpu.VMEM_SHARED((NPAD,), f32),
            pltpu.VMEM_SHARED((NPAD,), f32),
            pltpu.VMEM_SHARED((NPAD,), f32),
            pltpu.VMEM_SHARED((NPAD,), f32),
            pltpu.VMEM_SHARED((NPAD,), f32),
            pltpu.VMEM_SHARED((NPAD,), f32),
            pltpu.VMEM_SHARED((NPAD,), f32),
            pltpu.VMEM_SHARED((NPAD,), f32),
            pltpu.VMEM_SHARED((NPAD,), f32),
        ],
    )
    msg1, deg1, deg2, wn2 = sc1(row_p, col_p, w1e, w2e, xlt.reshape(5 * NPAD))
    msg1 = msg1.reshape(NC, 5, NPAD)
    deg1 = deg1[None, :]
    deg2 = deg2[None, :]

    xl2 = pl.pallas_call(
        _mid_body,
        in_specs=[
            _full_spec((NC, 5, NPAD)),
            _full_spec((1, NPAD)),
            _full_spec((5, NPAD)),
            _full_spec((1, NPAD)),
            _full_spec((5, 1)),
            _full_spec((5, 1)),
            _full_spec((5, 1)),
            _full_spec((5, 1)),
            _full_spec((5, 1)),
        ],
        out_specs=_full_spec((1, NPAD)),
        out_shape=jax.ShapeDtypeStruct((1, NPAD), f32),
    )(msg1, deg1, xlt, batch_p, b1[:, None], gn_w[:, None],
      gn_b[:, None], gn_ms[:, None], W2)

    sc2 = pl.kernel(
        _sc_msg2_body,
        out_type=[jax.ShapeDtypeStruct((NC * NPAD,), f32)],
        mesh=_sc_mesh(),
        compiler_params=pltpu.CompilerParams(needs_layout_passes=False),
        scratch_types=[
            pltpu.VMEM((ECHUNK,), jnp.int32),
            pltpu.VMEM((ECHUNK,), jnp.int32),
            pltpu.VMEM((ECHUNK,), f32),
            pltpu.VMEM((ECHUNK,), f32),
            pltpu.VMEM((NPAD,), f32),
            pltpu.VMEM((NPS,), f32),
            pltpu.VMEM_SHARED((NPAD,), f32),
        ],
    )
    (msg2,) = sc2(row_p, col_p, wn2, xl2.reshape(NPAD))
    msg2 = msg2.reshape(NC, NPAD)

    out = pl.pallas_call(
        _final_body,
        in_specs=[_full_spec((NC, NPAD)), _full_spec((1, NPAD)),
                  _full_spec((1, NPAD)), _full_spec((1, 1))],
        out_specs=_full_spec((1, NPAD)),
        out_shape=jax.ShapeDtypeStruct((1, NPAD), f32),
    )(msg2, deg2, xl2, b2[:, None])

    return out[0, :N, None]
